# baseline (speedup 1.0000x reference)
import functools

import jax
import jax.numpy as jnp
from jax import lax
from jax.experimental import pallas as pl
from jax.experimental.pallas import tpu as pltpu
from jax.experimental.pallas import tpu_sc as plsc

N = 10000
E = 320000
DF = 128

NCORE = 2
NSUB = 16
NT = NCORE * NSUB
CPT = 5
NCS = NT * CPT
CE = 2048
NCK = 160
E_PAD = NCK * CE
NG = CE // 16
GU = 4
NPAD = 10240
NZ16 = NPAD // 16

BLK = 1024


def _sc_body(cols_hbm, idxi_hbm, idxj_hbm, out_hbm,
             cols_v, iis, jjs, acc_v, isems, jsems):
    c = lax.axis_index("c")
    s = lax.axis_index("s")
    w = c * NSUB + s
    myrows = pl.ds(w * CPT, CPT)

    pltpu.sync_copy(cols_hbm.at[myrows], cols_v)

    def zbody(k, carry):
        z = jnp.zeros((16,), jnp.float32)
        for cl in range(CPT):
            acc_v[cl, pl.ds(k * 16, 16)] = z
        return carry

    lax.fori_loop(0, NZ16, zbody, 0)

    def issue(ck, b):
        sl = pl.ds(ck * CE, CE)
        pltpu.async_copy(idxi_hbm.at[sl], iis[b], isems[b])
        pltpu.async_copy(idxj_hbm.at[sl], jjs[b], jsems[b])

    def wait(b):
        sl = pl.ds(0, CE)
        pltpu.make_async_copy(idxi_hbm.at[sl], iis[b], isems[b]).wait()
        pltpu.make_async_copy(idxj_hbm.at[sl], jjs[b], jsems[b]).wait()

    for b in range(2):
        issue(b, b)

    def chunk_body(p, carry):
        for b in range(2):
            ck = p * 2 + b
            wait(b)

            def gbody(g, carry2):
                iiu, jju, vals = [], [], []
                for u in range(GU):
                    e0 = (g * GU + u) * 16
                    iiu.append(iis[b][pl.ds(e0, 16)])
                    jju.append(jjs[b][pl.ds(e0, 16)])
                for u in range(GU):
                    for cl in range(CPT):
                        clv = jnp.full((16,), cl, jnp.int32)
                        vals.append(plsc.load_gather(cols_v, [clv, jju[u]]))
                for u in range(GU):
                    for cl in range(CPT):
                        clv = jnp.full((16,), cl, jnp.int32)
                        plsc.addupdate_scatter(acc_v, [clv, iiu[u]],
                                               vals[u * CPT + cl])
                return carry2

            plsc.parallel_loop(0, NG // GU, 1, unroll=2)(
                lambda g: gbody(g, 0))

            @pl.when(ck + 2 < NCK)
            def _():
                issue(ck + 2, b)

        return carry

    lax.fori_loop(0, NCK // 2, chunk_body, 0)

    pltpu.sync_copy(acc_v, out_hbm.at[myrows])


@functools.cache
def _sc_segsum():
    return pl.kernel(
        _sc_body,
        out_type=jax.ShapeDtypeStruct((NCS, NPAD), jnp.float32),
        mesh=plsc.VectorSubcoreMesh(core_axis_name="c", subcore_axis_name="s"),
        compiler_params=pltpu.CompilerParams(use_tc_tiling_on_sc=False, needs_layout_passes=False),
        scratch_types=[
            pltpu.VMEM((CPT, N), jnp.float32),
            [pltpu.VMEM((CE,), jnp.int32) for _ in range(2)],
            [pltpu.VMEM((CE,), jnp.int32) for _ in range(2)],
            pltpu.VMEM((CPT, NPAD), jnp.float32),
            [pltpu.SemaphoreType.DMA for _ in range(2)],
            [pltpu.SemaphoreType.DMA for _ in range(2)],
        ],
    )


def _tc_dense_body(cols_ref, s_ref,
                   w1m_ref, b1m_ref, w1a_ref, b1a_ref,
                   wxi_ref, wdx_ref, wdp_ref, wctx_ref, b2m_ref,
                   w2ax_ref, w2aa_ref, b2a_ref, o_ref):
    f32 = jnp.float32

    def mm(a_ref, bT):
        return jnp.dot(a_ref[...], bT, preferred_element_type=f32)

    cols = cols_ref[...]
    xT = cols[0:DF]
    posT = cols[DF:DF + 3]
    st = s_ref[...]
    sxT = st[0:DF]
    spT = st[DF:DF + 3]
    cntT = st[DF + 3:DF + 4]
    invT = 1.0 / jnp.maximum(cntT, 1.0)
    nzT = cntT > 0.0
    mean_xT = sxT * invT
    dposT = spT * invT - posT
    aggr1T = jnp.where(nzT, mm(w1m_ref, dposT) + b1m_ref[...], 0.0)
    ctxT = mm(w1a_ref, aggr1T) + b1a_ref[...]
    aggr2T = jnp.where(
        nzT,
        mm(wxi_ref, xT) + mm(wdx_ref, mean_xT) + mm(wdp_ref, dposT)
        + mm(wctx_ref, ctxT) + b2m_ref[...],
        0.0)
    outT = mm(w2ax_ref, xT) + mm(w2aa_ref, aggr2T) + b2a_ref[...]
    o_ref[...] = outT.T


def _tc_dense(cols, sums, *weights, interpret=False):
    def _full(a):
        return pl.BlockSpec(a.shape, lambda i: (0,) * a.ndim)

    return pl.pallas_call(
        _tc_dense_body,
        grid=(pl.cdiv(N, BLK),),
        in_specs=[
            pl.BlockSpec((NCS, BLK), lambda i: (0, i)),
            pl.BlockSpec((NCS, BLK), lambda i: (0, i)),
        ] + [_full(a) for a in weights],
        out_specs=pl.BlockSpec((BLK, DF), lambda i: (i, 0)),
        out_shape=jax.ShapeDtypeStruct((N, DF), jnp.float32),
        interpret=interpret,
    )(cols, sums, *weights)


def kernel(x, edge_index, pos, W1m, b1m, W1a, b1a, W2m, b2m, W2a, b2a):
    idx_i = edge_index[0].astype(jnp.int32)
    idx_j = edge_index[1].astype(jnp.int32)
    cols = jnp.concatenate(
        [x.T, pos.T, jnp.ones((1, N), jnp.float32),
         jnp.zeros((NCS - DF - 4, N), jnp.float32)], axis=0)
    padn = E_PAD - E
    idx_i = jnp.concatenate([idx_i, jnp.full((padn,), N, jnp.int32)])
    idx_j = jnp.concatenate([idx_j, jnp.zeros((padn,), jnp.int32)])

    sums = _sc_segsum()(cols, idx_i, idx_j)

    wxiT = (W2m[0:DF] - W2m[DF:2 * DF]).T
    return _tc_dense(
        cols, sums,
        W1m.T, b1m.reshape(-1, 1), W1a.T, b1a.reshape(-1, 1),
        wxiT, W2m[DF:2 * DF].T, W2m[2 * DF:2 * DF + 3].T,
        W2m[2 * DF + 3:].T, b2m.reshape(-1, 1),
        W2a[:DF].T, W2a[DF:].T, b2a.reshape(-1, 1))


# speedup vs baseline: 1.0023x; 1.0023x over previous
import functools

import jax
import jax.numpy as jnp
from jax import lax
from jax.experimental import pallas as pl
from jax.experimental.pallas import tpu as pltpu
from jax.experimental.pallas import tpu_sc as plsc

N = 10000
E = 320000
DF = 128

NCORE = 2
NSUB = 16
NT = NCORE * NSUB
CPT = 5
NCS = NT * CPT
CE = 2048
NCK = 160
E_PAD = NCK * CE
NG = CE // 16
GU = 4
NPAD = 10240
NZ16 = NPAD // 16

BLK = 1024


def _sc_body(cols_hbm, idxi_hbm, idxj_hbm, out_hbm,
             cols_v, iis, jjs, acc_v, isems, jsems):
    c = lax.axis_index("c")
    s = lax.axis_index("s")
    w = c * NSUB + s
    myrows = pl.ds(w * CPT, CPT)

    pltpu.sync_copy(cols_hbm.at[myrows], cols_v)

    def zbody(k, carry):
        z = jnp.zeros((16,), jnp.float32)
        for cl in range(CPT):
            acc_v[cl, pl.ds(k * 16, 16)] = z
        return carry

    lax.fori_loop(0, NZ16, zbody, 0)

    def issue(ck, b):
        sl = pl.ds(ck * CE, CE)
        pltpu.async_copy(idxi_hbm.at[sl], iis[b], isems[b])
        pltpu.async_copy(idxj_hbm.at[sl], jjs[b], jsems[b])

    def wait(b):
        sl = pl.ds(0, CE)
        pltpu.make_async_copy(idxi_hbm.at[sl], iis[b], isems[b]).wait()
        pltpu.make_async_copy(idxj_hbm.at[sl], jjs[b], jsems[b]).wait()

    for b in range(2):
        issue(b, b)

    def chunk_body(p, carry):
        for b in range(2):
            ck = p * 2 + b
            wait(b)

            def gbody(g, carry2):
                iiu, jju, vals = [], [], []
                for u in range(GU):
                    e0 = (g * GU + u) * 16
                    iiu.append(iis[b][pl.ds(e0, 16)])
                    jju.append(jjs[b][pl.ds(e0, 16)])
                for u in range(GU):
                    for cl in range(CPT):
                        clv = jnp.full((16,), cl, jnp.int32)
                        vals.append(plsc.load_gather(cols_v, [clv, jju[u]]))
                for u in range(GU):
                    for cl in range(CPT):
                        clv = jnp.full((16,), cl, jnp.int32)
                        plsc.addupdate_scatter(acc_v, [clv, iiu[u]],
                                               vals[u * CPT + cl])
                return carry2

            plsc.parallel_loop(0, NG // GU, 1, unroll=1)(
                lambda g: gbody(g, 0))

            @pl.when(ck + 2 < NCK)
            def _():
                issue(ck + 2, b)

        return carry

    lax.fori_loop(0, NCK // 2, chunk_body, 0)

    pltpu.sync_copy(acc_v, out_hbm.at[myrows])


@functools.cache
def _sc_segsum():
    return pl.kernel(
        _sc_body,
        out_type=jax.ShapeDtypeStruct((NCS, NPAD), jnp.float32),
        mesh=plsc.VectorSubcoreMesh(core_axis_name="c", subcore_axis_name="s"),
        compiler_params=pltpu.CompilerParams(use_tc_tiling_on_sc=False, needs_layout_passes=False),
        scratch_types=[
            pltpu.VMEM((CPT, N), jnp.float32),
            [pltpu.VMEM((CE,), jnp.int32) for _ in range(2)],
            [pltpu.VMEM((CE,), jnp.int32) for _ in range(2)],
            pltpu.VMEM((CPT, NPAD), jnp.float32),
            [pltpu.SemaphoreType.DMA for _ in range(2)],
            [pltpu.SemaphoreType.DMA for _ in range(2)],
        ],
    )


def _tc_dense_body(cols_ref, s_ref,
                   w1m_ref, b1m_ref, w1a_ref, b1a_ref,
                   wxi_ref, wdx_ref, wdp_ref, wctx_ref, b2m_ref,
                   w2ax_ref, w2aa_ref, b2a_ref, o_ref):
    f32 = jnp.float32

    def mm(a_ref, bT):
        return jnp.dot(a_ref[...], bT, preferred_element_type=f32)

    cols = cols_ref[...]
    xT = cols[0:DF]
    posT = cols[DF:DF + 3]
    st = s_ref[...]
    sxT = st[0:DF]
    spT = st[DF:DF + 3]
    cntT = st[DF + 3:DF + 4]
    invT = 1.0 / jnp.maximum(cntT, 1.0)
    nzT = cntT > 0.0
    mean_xT = sxT * invT
    dposT = spT * invT - posT
    aggr1T = jnp.where(nzT, mm(w1m_ref, dposT) + b1m_ref[...], 0.0)
    ctxT = mm(w1a_ref, aggr1T) + b1a_ref[...]
    aggr2T = jnp.where(
        nzT,
        mm(wxi_ref, xT) + mm(wdx_ref, mean_xT) + mm(wdp_ref, dposT)
        + mm(wctx_ref, ctxT) + b2m_ref[...],
        0.0)
    outT = mm(w2ax_ref, xT) + mm(w2aa_ref, aggr2T) + b2a_ref[...]
    o_ref[...] = outT.T


def _tc_dense(cols, sums, *weights, interpret=False):
    def _full(a):
        return pl.BlockSpec(a.shape, lambda i: (0,) * a.ndim)

    return pl.pallas_call(
        _tc_dense_body,
        grid=(pl.cdiv(N, BLK),),
        in_specs=[
            pl.BlockSpec((NCS, BLK), lambda i: (0, i)),
            pl.BlockSpec((NCS, BLK), lambda i: (0, i)),
        ] + [_full(a) for a in weights],
        out_specs=pl.BlockSpec((BLK, DF), lambda i: (i, 0)),
        out_shape=jax.ShapeDtypeStruct((N, DF), jnp.float32),
        interpret=interpret,
    )(cols, sums, *weights)


def kernel(x, edge_index, pos, W1m, b1m, W1a, b1a, W2m, b2m, W2a, b2a):
    idx_i = edge_index[0].astype(jnp.int32)
    idx_j = edge_index[1].astype(jnp.int32)
    cols = jnp.concatenate(
        [x.T, pos.T, jnp.ones((1, N), jnp.float32),
         jnp.zeros((NCS - DF - 4, N), jnp.float32)], axis=0)
    padn = E_PAD - E
    idx_i = jnp.concatenate([idx_i, jnp.full((padn,), N, jnp.int32)])
    idx_j = jnp.concatenate([idx_j, jnp.zeros((padn,), jnp.int32)])

    sums = _sc_segsum()(cols, idx_i, idx_j)

    wxiT = (W2m[0:DF] - W2m[DF:2 * DF]).T
    return _tc_dense(
        cols, sums,
        W1m.T, b1m.reshape(-1, 1), W1a.T, b1a.reshape(-1, 1),
        wxiT, W2m[DF:2 * DF].T, W2m[2 * DF:2 * DF + 3].T,
        W2m[2 * DF + 3:].T, b2m.reshape(-1, 1),
        W2a[:DF].T, W2a[DF:].T, b2a.reshape(-1, 1))
